# baseline (speedup 1.0000x reference)
import functools

import jax
import jax.numpy as jnp
import numpy as np
from jax import lax
from jax.experimental import pallas as pl
from jax.experimental.pallas import tpu as pltpu
from jax.experimental.pallas import tpu_sc as plsc

B = 16
N = 2048
S = 512
K = 32
CIN = 128
DTAB = 256
R2 = np.float32(0.2 ** 2)
F32 = jnp.float32
HI = jax.lax.Precision.HIGHEST


def _fps_kernel(xyz_ref, out_ref):
    xq = xyz_ref[0]
    yq = xyz_ref[1]
    zq = xyz_ref[2]
    iota_n = lax.broadcasted_iota(jnp.int32, (B, N), 1).astype(F32)
    iota_s = lax.broadcasted_iota(jnp.int32, (B, S), 1).astype(F32)

    def body(i, carry):
        dist, far, ax, ay, az = carry
        sel = iota_n == far
        cx = jnp.sum(jnp.where(sel, xq, 0.0), axis=1, keepdims=True)
        cy = jnp.sum(jnp.where(sel, yq, 0.0), axis=1, keepdims=True)
        cz = jnp.sum(jnp.where(sel, zq, 0.0), axis=1, keepdims=True)
        i_f = i.astype(F32)
        ax = jnp.where(iota_s == i_f, cx, ax)
        ay = jnp.where(iota_s == i_f, cy, ay)
        az = jnp.where(iota_s == i_f, cz, az)
        dx = xq - cx
        dy = yq - cy
        dz = zq - cz
        d = dx * dx + dy * dy + dz * dz
        dist = jnp.minimum(dist, d)
        m = jnp.max(dist, axis=1, keepdims=True)
        far = jnp.min(jnp.where(dist == m, iota_n, float(N)), axis=1,
                      keepdims=True)
        return dist, far, ax, ay, az

    dist0 = jnp.full((B, N), 1e10, dtype=F32)
    far0 = jnp.zeros((B, 1), dtype=F32)
    acc0 = jnp.zeros((B, S), dtype=F32)
    _, _, ax, ay, az = lax.fori_loop(0, S, body,
                                     (dist0, far0, acc0, acc0, acc0))
    out_ref[0] = ax
    out_ref[1] = ay
    out_ref[2] = az


def _fps_call(xyz_t):
    return pl.pallas_call(
        _fps_kernel,
        out_shape=jax.ShapeDtypeStruct((3, B, S), F32),
    )(xyz_t)


S_BLK = 64
_CHUNK = 256


def _bq_kernel(xyz_ref, xyzr_ref, nxyz_ref, out_ref, gxx_ref, gxy_ref,
               gxz_ref):
    b = pl.program_id(0)
    xq = xyz_ref[0, 0:1, :]
    yq = xyz_ref[0, 1:2, :]
    zq = xyz_ref[0, 2:3, :]
    nb = nxyz_ref[0]
    ax = nb[:, 0:1]
    ay = nb[:, 1:2]
    az = nb[:, 2:3]
    dot = lax.dot_general(nb.astype(jnp.bfloat16),
                          xyz_ref[0].astype(jnp.bfloat16),
                          (((1,), (0,)), ((), ())),
                          preferred_element_type=F32)
    ns = ax * ax + ay * ay + az * az
    nx = xq * xq + yq * yq + zq * zq
    sqr = (ns + nx) - 2.0 * dot
    mask = jnp.where(sqr <= R2, 1.0, 0.0)

    cnt = mask
    lane = lax.broadcasted_iota(jnp.int32, (S_BLK, N), 1).astype(F32)
    sh = 1
    while sh < N:
        rolled = jnp.roll(cnt, sh, axis=1)
        cnt = cnt + jnp.where(lane >= float(sh), rolled, 0.0)
        sh *= 2

    total = cnt[:, N - 1:N]
    capped = jnp.minimum(cnt, 33.0)
    kv1 = (lax.broadcasted_iota(jnp.int32, (1, K, 1), 1) + 1).astype(F32)
    iota_col = lax.broadcasted_iota(jnp.int32, (N, 1), 0).astype(F32)
    vals = jnp.concatenate(
        [iota_col, xyzr_ref[0], jnp.zeros((N, 4), F32)], axis=1)
    acc8 = jnp.zeros((S_BLK * K, 8), dtype=F32)
    for c in range(N // _CHUNK):
        sl = slice(c * _CHUNK, (c + 1) * _CHUNK)
        delta = (jnp.where(capped[:, None, sl] == kv1, 1.0, 0.0)
                 * mask[:, None, sl]).reshape(S_BLK * K, _CHUNK)
        acc8 = acc8 + lax.dot_general(delta, vals[sl, :],
                                      (((1,), (0,)), ((), ())), precision=HI)
    acc = acc8.reshape(S_BLK, K, 8)
    pos = acc[:, :, 0:1]
    krow = lax.broadcasted_iota(jnp.int32, (S_BLK, K, 1), 1).astype(F32)
    valid = krow < total.reshape(S_BLK, 1, 1)
    idx = jnp.where(valid, pos, pos[:, 0:1, :])
    gidx = idx + b.astype(F32) * float(N)
    out_ref[0] = gidx.astype(jnp.int32)
    gxx_ref[0] = jnp.where(valid, acc[:, :, 1:2], acc[:, 0:1, 1:2])
    gxy_ref[0] = jnp.where(valid, acc[:, :, 2:3], acc[:, 0:1, 2:3])
    gxz_ref[0] = jnp.where(valid, acc[:, :, 3:4], acc[:, 0:1, 3:4])


def _bq_call(xyz_bt, xyz, new_xyz):
    return pl.pallas_call(
        _bq_kernel,
        grid=(B, S // S_BLK),
        in_specs=[
            pl.BlockSpec((1, 3, N), lambda b, s: (b, 0, 0)),
            pl.BlockSpec((1, N, 3), lambda b, s: (b, 0, 0)),
            pl.BlockSpec((1, S_BLK, 3), lambda b, s: (b, s, 0)),
        ],
        out_specs=[
            pl.BlockSpec((1, S_BLK, K, 1), lambda b, s: (b, s, 0, 0)),
            pl.BlockSpec((1, S_BLK, K, 1), lambda b, s: (b, s, 0, 0)),
            pl.BlockSpec((1, S_BLK, K, 1), lambda b, s: (b, s, 0, 0)),
            pl.BlockSpec((1, S_BLK, K, 1), lambda b, s: (b, s, 0, 0)),
        ],
        out_shape=[
            jax.ShapeDtypeStruct((B, S, K, 1), jnp.int32),
            jax.ShapeDtypeStruct((B, S, K, 1), F32),
            jax.ShapeDtypeStruct((B, S, K, 1), F32),
            jax.ShapeDtypeStruct((B, S, K, 1), F32),
        ],
        compiler_params=pltpu.CompilerParams(
            dimension_semantics=("arbitrary", "arbitrary")),
    )(xyz_bt, xyz, new_xyz)


_GROWS = B * S * K
_GCH = 128


def _gather_sc(table_hbm, gidx_hbm, out_hbm, idx_v, rows0, rows1, sem0, sem1,
               *, rows_per_worker, n_chunks):
    wid = lax.axis_index("s") * 2 + lax.axis_index("c")
    base = wid * rows_per_worker
    pltpu.sync_copy(gidx_hbm.at[pl.ds(base, rows_per_worker)], idx_v)
    bufs = (rows0, rows1)
    sems = (sem0, sem1)
    pltpu.async_copy(table_hbm.at[idx_v.at[pl.ds(0, _GCH)]], rows0, sem0)

    def pair(p, _):
        for ph in range(2):
            j = p * 2 + ph
            buf = bufs[ph]
            pltpu.make_async_copy(table_hbm.at[idx_v.at[pl.ds(0, _GCH)]],
                                  buf, sems[ph]).wait()

            @pl.when(j + 1 < n_chunks)
            def _():
                nxt = (j + 1) * _GCH
                pltpu.async_copy(
                    table_hbm.at[idx_v.at[pl.ds(nxt, _GCH)]],
                    bufs[1 - ph], sems[1 - ph])

            pltpu.sync_copy(buf, out_hbm.at[pl.ds(base + j * _GCH, _GCH)])
        return 0

    lax.fori_loop(0, n_chunks // 2, pair, 0)


def _gather_call(table, gidx):
    info = plsc.get_sparse_core_info()
    nw = info.num_cores * info.num_subcores
    rows_per_worker = _GROWS // nw
    n_chunks = rows_per_worker // _GCH
    mesh = plsc.VectorSubcoreMesh(core_axis_name="c", subcore_axis_name="s")
    f = functools.partial(_gather_sc, rows_per_worker=rows_per_worker,
                          n_chunks=n_chunks)
    return pl.kernel(
        f,
        mesh=mesh,
        out_type=jax.ShapeDtypeStruct((_GROWS, CIN), F32),
        scratch_types=[
            pltpu.VMEM((rows_per_worker,), jnp.int32),
            pltpu.VMEM((_GCH, CIN), F32),
            pltpu.VMEM((_GCH, CIN), F32),
            pltpu.SemaphoreType.DMA,
            pltpu.SemaphoreType.DMA,
        ],
    )(table, gidx)


LT = 2048
NTILES = (B * S * K) // LT
TPB = (S * K) // LT
NTOT = float(B * S * K)


_SQRT_HALF = np.float32(1.0 / np.sqrt(2.0))


def _gelu(x):
    return 0.5 * x * (1.0 + lax.erf(x * _SQRT_HALF))


def _tbias(te_ref, tw_ref, tb_ref):
    t_act = _gelu(te_ref[0])
    return lax.dot_general(t_act, tw_ref[...], (((1,), (1,)), ((), ())),
                           precision=HI) + tb_ref[...]


def _bn_apply(x, st_ref, g_ref, b_ref):
    mu = st_ref[0:1, :] * (1.0 / NTOT)
    ex2 = st_ref[1:2, :] * (1.0 / NTOT)
    var = ex2 - mu * mu
    denom = jnp.sqrt(var + 1e-5)
    return ((x - mu) / denom) * g_ref[...] + b_ref[...]


def _acc_stats(st_ref, y):
    @pl.when((pl.program_id(0) == 0) & (pl.program_id(1) == 0))
    def _():
        st_ref[...] = jnp.zeros_like(st_ref)

    st_ref[0:1, :] += jnp.sum(y, axis=0, keepdims=True)
    st_ref[1:2, :] += jnp.sum(y * y, axis=0, keepdims=True)


def _conv1_kernel(g_ref, g3_ref, nx_ref, te_ref, tw_ref, tb_ref, wf_ref,
                  wxyz_ref, y_ref, st_ref):
    feat = g_ref[...]
    y = lax.dot_general(feat, wf_ref[...], (((1,), (1,)), ((), ())),
                        precision=HI)
    wxyz = wxyz_ref[...]
    y = y + lax.dot_general(g3_ref[...], wxyz, (((1,), (1,)), ((), ())),
                            precision=HI)
    corr = lax.dot_general(nx_ref[0], wxyz, (((1,), (1,)), ((), ())),
                           precision=HI)
    y = (y.reshape(LT // K, K, 128) - corr[:, None, :]).reshape(LT, 128)
    y = y + _tbias(te_ref, tw_ref, tb_ref)
    _acc_stats(st_ref, y)
    y_ref[0] = y


def _convn_kernel(x_ref, stp_ref, g_ref, b_ref, te_ref, tw_ref, tb_ref,
                  w_ref, y_ref, st_ref):
    x = _gelu(_bn_apply(x_ref[0], stp_ref, g_ref, b_ref))
    y = lax.dot_general(x, w_ref[...], (((1,), (1,)), ((), ())), precision=HI)
    y = y + _tbias(te_ref, tw_ref, tb_ref)
    _acc_stats(st_ref, y)
    y_ref[0] = y


def _pool_kernel(x_ref, stp_ref, g_ref, b_ref, out_ref):
    x = _gelu(_bn_apply(x_ref[0], stp_ref, g_ref, b_ref))
    out_ref[0] = jnp.max(x.reshape(LT // K, K, 256), axis=1)


def _row_spec(c):
    return pl.BlockSpec((1, LT, c), lambda b, l: (b, l, 0))


def _full_spec(shape):
    nd = len(shape)
    return pl.BlockSpec(shape, lambda b, l: (0,) * nd)


def _te_spec():
    return pl.BlockSpec((1, 1, 256), lambda b, l: (b, 0, 0))


def _params(outs=1):
    sem = ("arbitrary", "arbitrary")
    return dict(compiler_params=pltpu.CompilerParams(
        dimension_semantics=sem))


def _conv1_call(gathered, gxyz, new_xyz, te, tw, tb, wf, wxyz):
    return pl.pallas_call(
        _conv1_kernel,
        grid=(B, TPB),
        in_specs=[
            pl.BlockSpec((LT, CIN), lambda b, l: (b * TPB + l, 0)),
            pl.BlockSpec((LT, 3), lambda b, l: (b * TPB + l, 0)),
            pl.BlockSpec((1, LT // K, 3), lambda b, l: (b, l, 0)),
            _te_spec(),
            _full_spec((128, 256)),
            _full_spec((1, 128)),
            _full_spec((128, CIN)),
            _full_spec((128, 3)),
        ],
        out_specs=[_row_spec(128), _full_spec((2, 128))],
        out_shape=[
            jax.ShapeDtypeStruct((B, S * K, 128), F32),
            jax.ShapeDtypeStruct((2, 128), F32),
        ],
        **_params(),
    )(gathered, gxyz, new_xyz, te, tw, tb, wf, wxyz)


def _convn_call(kfn, x, stp, g, bb, te, tw, tb, w, oc):
    cin = x.shape[-1]
    return pl.pallas_call(
        kfn,
        grid=(B, TPB),
        in_specs=[
            _row_spec(cin),
            _full_spec((2, cin)),
            _full_spec((1, cin)),
            _full_spec((1, cin)),
            _te_spec(),
            _full_spec((oc, 256)),
            _full_spec((1, oc)),
            _full_spec((oc, cin)),
        ],
        out_specs=[_row_spec(oc), _full_spec((2, oc))],
        out_shape=[
            jax.ShapeDtypeStruct((B, S * K, oc), F32),
            jax.ShapeDtypeStruct((2, oc), F32),
        ],
        **_params(),
    )(x, stp, g, bb, te, tw, tb, w)


def _pool_call(x, stp, g, bb):
    return pl.pallas_call(
        _pool_kernel,
        grid=(B, TPB),
        in_specs=[
            _row_spec(256),
            _full_spec((2, 256)),
            _full_spec((1, 256)),
            _full_spec((1, 256)),
        ],
        out_specs=pl.BlockSpec((1, LT // K, 256), lambda b, l: (b, l, 0)),
        out_shape=jax.ShapeDtypeStruct((B, S, 256), F32),
        **_params(),
    )(x, stp, g, bb)


def kernel(xyz, points, t_embed, conv_w_0, time_w_0, time_b_0, bn_g_0,
           bn_b_0, conv_w_1, time_w_1, time_b_1, bn_g_1, bn_b_1, conv_w_2,
           time_w_2, time_b_2, bn_g_2, bn_b_2):
    xyz_t = jnp.transpose(xyz, (2, 0, 1))
    nx3 = _fps_call(xyz_t)
    new_xyz = jnp.transpose(nx3, (1, 2, 0))

    xyz_bt = jnp.transpose(xyz, (0, 2, 1))
    gidx, gxx, gxy, gxz = _bq_call(xyz_bt, xyz, new_xyz)
    gidx_flat = gidx.reshape(_GROWS)
    gxyz = jnp.concatenate([gxx, gxy, gxz], axis=-1).reshape(_GROWS, 3)

    pts_t = jnp.transpose(points, (0, 2, 1))
    table = pts_t.reshape(B * N, CIN)
    gathered = _gather_call(table, gidx_flat)

    te = t_embed.reshape(B, 1, 256)
    y1, st1 = _conv1_call(gathered, gxyz, new_xyz, te, time_w_0,
                          time_b_0.reshape(1, 128), conv_w_0[:, 3:],
                          conv_w_0[:, :3])
    y2, st2 = _convn_call(_convn_kernel, y1, st1, bn_g_0.reshape(1, 128),
                          bn_b_0.reshape(1, 128), te, time_w_1,
                          time_b_1.reshape(1, 128), conv_w_1, 128)
    y3, st3 = _convn_call(_convn_kernel, y2, st2, bn_g_1.reshape(1, 128),
                          bn_b_1.reshape(1, 128), te, time_w_2,
                          time_b_2.reshape(1, 256), conv_w_2, 256)
    pooled = _pool_call(y3, st3, bn_g_2.reshape(1, 256),
                        bn_b_2.reshape(1, 256))
    new_points = jnp.transpose(pooled, (0, 2, 1))
    return new_xyz, new_points


# speedup vs baseline: 2.4661x; 2.4661x over previous
import functools

import jax
import jax.numpy as jnp
import numpy as np
from jax import lax
from jax.experimental import pallas as pl
from jax.experimental.pallas import tpu as pltpu
from jax.experimental.pallas import tpu_sc as plsc

B = 16
N = 2048
S = 512
K = 32
CIN = 128
DTAB = 256
R2 = np.float32(0.2 ** 2)
F32 = jnp.float32
HI = jax.lax.Precision.HIGHEST


def _fps_kernel(xyz_ref, out_ref):
    xq = xyz_ref[0]
    yq = xyz_ref[1]
    zq = xyz_ref[2]
    iota_n = lax.broadcasted_iota(jnp.int32, (B, N), 1).astype(F32)
    iota_s = lax.broadcasted_iota(jnp.int32, (B, S), 1).astype(F32)

    def body(i, carry):
        dist, far, ax, ay, az = carry
        sel = iota_n == far
        cx = jnp.sum(jnp.where(sel, xq, 0.0), axis=1, keepdims=True)
        cy = jnp.sum(jnp.where(sel, yq, 0.0), axis=1, keepdims=True)
        cz = jnp.sum(jnp.where(sel, zq, 0.0), axis=1, keepdims=True)
        i_f = i.astype(F32)
        ax = jnp.where(iota_s == i_f, cx, ax)
        ay = jnp.where(iota_s == i_f, cy, ay)
        az = jnp.where(iota_s == i_f, cz, az)
        dx = xq - cx
        dy = yq - cy
        dz = zq - cz
        d = dx * dx + dy * dy + dz * dz
        dist = jnp.minimum(dist, d)
        m = jnp.max(dist, axis=1, keepdims=True)
        far = jnp.min(jnp.where(dist == m, iota_n, float(N)), axis=1,
                      keepdims=True)
        return dist, far, ax, ay, az

    dist0 = jnp.full((B, N), 1e10, dtype=F32)
    far0 = jnp.zeros((B, 1), dtype=F32)
    acc0 = jnp.zeros((B, S), dtype=F32)
    _, _, ax, ay, az = lax.fori_loop(0, S, body,
                                     (dist0, far0, acc0, acc0, acc0))
    out_ref[0] = ax
    out_ref[1] = ay
    out_ref[2] = az


def _fps_call(xyz_t):
    return pl.pallas_call(
        _fps_kernel,
        out_shape=jax.ShapeDtypeStruct((3, B, S), F32),
    )(xyz_t)


S_BLK = 64
_CHUNK = 256


def _bq_kernel(xyz_ref, nxyz_ref, out_ref):
    b = pl.program_id(0)
    xq = xyz_ref[0, 0:1, :]
    yq = xyz_ref[0, 1:2, :]
    zq = xyz_ref[0, 2:3, :]
    nb = nxyz_ref[0]
    ax = nb[:, 0:1]
    ay = nb[:, 1:2]
    az = nb[:, 2:3]
    dot = lax.dot_general(nb.astype(jnp.bfloat16),
                          xyz_ref[0].astype(jnp.bfloat16),
                          (((1,), (0,)), ((), ())),
                          preferred_element_type=F32)
    ns = ax * ax + ay * ay + az * az
    nx = xq * xq + yq * yq + zq * zq
    sqr = (ns + nx) - 2.0 * dot
    mask = jnp.where(sqr <= R2, 1.0, 0.0)

    cnt = mask
    lane = lax.broadcasted_iota(jnp.int32, (S_BLK, N), 1).astype(F32)
    sh = 1
    while sh < N:
        rolled = jnp.roll(cnt, sh, axis=1)
        cnt = cnt + jnp.where(lane >= float(sh), rolled, 0.0)
        sh *= 2

    total = cnt[:, N - 1:N]
    capped = jnp.minimum(cnt, 32.0)
    kv = lax.broadcasted_iota(jnp.int32, (1, K, 1), 1).astype(F32)
    acc = jnp.zeros((S_BLK, K), dtype=F32)
    for c in range(N // _CHUNK):
        sub = capped[:, c * _CHUNK:(c + 1) * _CHUNK]
        cmp = jnp.where(sub[:, None, :] <= kv, 1.0, 0.0)
        acc = acc + jnp.sum(cmp, axis=2)
    krow = lax.broadcasted_iota(jnp.int32, (S_BLK, K), 1).astype(F32)
    first = acc[:, 0:1]
    idx = jnp.where(krow < total, acc, first)
    gidx = idx + b.astype(F32) * float(N)
    out_ref[0] = gidx.astype(jnp.int32)


def _bq_call(xyz_bt, new_xyz):
    return pl.pallas_call(
        _bq_kernel,
        grid=(B, S // S_BLK),
        in_specs=[
            pl.BlockSpec((1, 3, N), lambda b, s: (b, 0, 0)),
            pl.BlockSpec((1, S_BLK, 3), lambda b, s: (b, s, 0)),
        ],
        out_specs=pl.BlockSpec((1, S_BLK, K), lambda b, s: (b, s, 0)),
        out_shape=jax.ShapeDtypeStruct((B, S, K), jnp.int32),
        compiler_params=pltpu.CompilerParams(
            dimension_semantics=("arbitrary", "arbitrary")),
    )(xyz_bt, new_xyz)


_GROWS = B * S * K
_GCH = 128


def _gather_sc(table_hbm, gidx_hbm, out_hbm, idx_v, rows0, rows1, sem0, sem1,
               *, rows_per_worker, n_chunks):
    wid = lax.axis_index("s") * 2 + lax.axis_index("c")
    base = wid * rows_per_worker
    pltpu.sync_copy(gidx_hbm.at[pl.ds(base, rows_per_worker)], idx_v)
    bufs = (rows0, rows1)
    sems = (sem0, sem1)
    pltpu.async_copy(table_hbm.at[idx_v.at[pl.ds(0, _GCH)]], rows0, sem0)

    def pair(p, _):
        for ph in range(2):
            j = p * 2 + ph
            buf = bufs[ph]
            pltpu.make_async_copy(table_hbm.at[idx_v.at[pl.ds(0, _GCH)]],
                                  buf, sems[ph]).wait()

            @pl.when(j + 1 < n_chunks)
            def _():
                nxt = (j + 1) * _GCH
                pltpu.async_copy(
                    table_hbm.at[idx_v.at[pl.ds(nxt, _GCH)]],
                    bufs[1 - ph], sems[1 - ph])

            pltpu.sync_copy(buf, out_hbm.at[pl.ds(base + j * _GCH, _GCH)])
        return 0

    lax.fori_loop(0, n_chunks // 2, pair, 0)


def _gather_call(table, gidx):
    info = plsc.get_sparse_core_info()
    nw = info.num_cores * info.num_subcores
    rows_per_worker = _GROWS // nw
    n_chunks = rows_per_worker // _GCH
    mesh = plsc.VectorSubcoreMesh(core_axis_name="c", subcore_axis_name="s")
    f = functools.partial(_gather_sc, rows_per_worker=rows_per_worker,
                          n_chunks=n_chunks)
    return pl.kernel(
        f,
        mesh=mesh,
        out_type=jax.ShapeDtypeStruct((_GROWS, DTAB), F32),
        scratch_types=[
            pltpu.VMEM((rows_per_worker,), jnp.int32),
            pltpu.VMEM((_GCH, DTAB), F32),
            pltpu.VMEM((_GCH, DTAB), F32),
            pltpu.SemaphoreType.DMA,
            pltpu.SemaphoreType.DMA,
        ],
    )(table, gidx)


LT = 2048
NTILES = (B * S * K) // LT
TPB = (S * K) // LT
NTOT = float(B * S * K)


_SQRT_HALF = np.float32(1.0 / np.sqrt(2.0))


def _gelu(x):
    return 0.5 * x * (1.0 + lax.erf(x * _SQRT_HALF))


def _tbias(te_ref, tw_ref, tb_ref):
    t_act = _gelu(te_ref[0])
    return lax.dot_general(t_act, tw_ref[...],
                           (((1,), (1,)), ((), ()))) + tb_ref[...]


def _bn_apply(x, st_ref, g_ref, b_ref):
    mu = st_ref[0:1, :] * (1.0 / NTOT)
    ex2 = st_ref[1:2, :] * (1.0 / NTOT)
    var = ex2 - mu * mu
    denom = jnp.sqrt(var + 1e-5)
    return ((x - mu) / denom) * g_ref[...] + b_ref[...]


def _acc_stats(st_ref, y):
    @pl.when((pl.program_id(0) == 0) & (pl.program_id(1) == 0))
    def _():
        st_ref[...] = jnp.zeros_like(st_ref)

    st_ref[0:1, :] += jnp.sum(y, axis=0, keepdims=True)
    st_ref[1:2, :] += jnp.sum(y * y, axis=0, keepdims=True)


def _conv1_kernel(g_ref, nx_ref, te_ref, tw_ref, tb_ref, w_ref,
                  y_ref, st_ref):
    feat = g_ref[...]
    w = w_ref[...]
    y = lax.dot_general(feat, w, (((1,), (1,)), ((), ())))
    wxyz = w[:, 0:3]
    corr = lax.dot_general(nx_ref[0], wxyz, (((1,), (1,)), ((), ())),
                           precision=HI)
    y = (y.reshape(LT // K, K, 128) - corr[:, None, :]).reshape(LT, 128)
    y = y + _tbias(te_ref, tw_ref, tb_ref)
    _acc_stats(st_ref, y)
    y_ref[0] = y


def _convn_kernel(x_ref, stp_ref, g_ref, b_ref, te_ref, tw_ref, tb_ref,
                  w_ref, y_ref, st_ref):
    x = _gelu(_bn_apply(x_ref[0], stp_ref, g_ref, b_ref))
    y = lax.dot_general(x, w_ref[...], (((1,), (1,)), ((), ())))
    y = y + _tbias(te_ref, tw_ref, tb_ref)
    _acc_stats(st_ref, y)
    y_ref[0] = y


def _pool_kernel(x_ref, stp_ref, g_ref, b_ref, out_ref):
    x = _gelu(_bn_apply(x_ref[0], stp_ref, g_ref, b_ref))
    out_ref[0] = jnp.max(x.reshape(LT // K, K, 256), axis=1)


def _row_spec(c):
    return pl.BlockSpec((1, LT, c), lambda b, l: (b, l, 0))


def _full_spec(shape):
    nd = len(shape)
    return pl.BlockSpec(shape, lambda b, l: (0,) * nd)


def _te_spec():
    return pl.BlockSpec((1, 1, 256), lambda b, l: (b, 0, 0))


def _params(outs=1):
    sem = ("arbitrary", "arbitrary")
    return dict(compiler_params=pltpu.CompilerParams(
        dimension_semantics=sem))


def _conv1_call(gathered, new_xyz, te, tw, tb, w):
    return pl.pallas_call(
        _conv1_kernel,
        grid=(B, TPB),
        in_specs=[
            pl.BlockSpec((LT, DTAB), lambda b, l: (b * TPB + l, 0)),
            pl.BlockSpec((1, LT // K, 3), lambda b, l: (b, l, 0)),
            _te_spec(),
            _full_spec((128, 256)),
            _full_spec((1, 128)),
            _full_spec((128, DTAB)),
        ],
        out_specs=[_row_spec(128), _full_spec((2, 128))],
        out_shape=[
            jax.ShapeDtypeStruct((B, S * K, 128), F32),
            jax.ShapeDtypeStruct((2, 128), F32),
        ],
        **_params(),
    )(gathered, new_xyz, te, tw, tb, w)


def _convn_call(kfn, x, stp, g, bb, te, tw, tb, w, oc):
    cin = x.shape[-1]
    return pl.pallas_call(
        kfn,
        grid=(B, TPB),
        in_specs=[
            _row_spec(cin),
            _full_spec((2, cin)),
            _full_spec((1, cin)),
            _full_spec((1, cin)),
            _te_spec(),
            _full_spec((oc, 256)),
            _full_spec((1, oc)),
            _full_spec((oc, cin)),
        ],
        out_specs=[_row_spec(oc), _full_spec((2, oc))],
        out_shape=[
            jax.ShapeDtypeStruct((B, S * K, oc), F32),
            jax.ShapeDtypeStruct((2, oc), F32),
        ],
        **_params(),
    )(x, stp, g, bb, te, tw, tb, w)


def _pool_call(x, stp, g, bb):
    return pl.pallas_call(
        _pool_kernel,
        grid=(B, TPB),
        in_specs=[
            _row_spec(256),
            _full_spec((2, 256)),
            _full_spec((1, 256)),
            _full_spec((1, 256)),
        ],
        out_specs=pl.BlockSpec((1, LT // K, 256), lambda b, l: (b, l, 0)),
        out_shape=jax.ShapeDtypeStruct((B, S, 256), F32),
        **_params(),
    )(x, stp, g, bb)


def kernel(xyz, points, t_embed, conv_w_0, time_w_0, time_b_0, bn_g_0,
           bn_b_0, conv_w_1, time_w_1, time_b_1, bn_g_1, bn_b_1, conv_w_2,
           time_w_2, time_b_2, bn_g_2, bn_b_2):
    xyz_t = jnp.transpose(xyz, (2, 0, 1))
    nx3 = _fps_call(xyz_t)
    new_xyz = jnp.transpose(nx3, (1, 2, 0))

    xyz_bt = jnp.transpose(xyz, (0, 2, 1))
    gidx = _bq_call(xyz_bt, new_xyz)
    gidx_flat = gidx.reshape(_GROWS)

    pts_t = jnp.transpose(points, (0, 2, 1))
    table = jnp.concatenate(
        [xyz, pts_t, jnp.zeros((B, N, DTAB - 3 - CIN), F32)],
        axis=-1).reshape(B * N, DTAB)
    gathered = _gather_call(table, gidx_flat)

    te = t_embed.reshape(B, 1, 256)
    w1 = jnp.concatenate([conv_w_0, jnp.zeros((128, DTAB - 131), F32)], 1)
    y1, st1 = _conv1_call(gathered, new_xyz, te, time_w_0,
                          time_b_0.reshape(1, 128), w1)
    y2, st2 = _convn_call(_convn_kernel, y1, st1, bn_g_0.reshape(1, 128),
                          bn_b_0.reshape(1, 128), te, time_w_1,
                          time_b_1.reshape(1, 128), conv_w_1, 128)
    y3, st3 = _convn_call(_convn_kernel, y2, st2, bn_g_1.reshape(1, 128),
                          bn_b_1.reshape(1, 128), te, time_w_2,
                          time_b_2.reshape(1, 256), conv_w_2, 256)
    pooled = _pool_call(y3, st3, bn_g_2.reshape(1, 256),
                        bn_b_2.reshape(1, 256))
    new_points = jnp.transpose(pooled, (0, 2, 1))
    return new_xyz, new_points


# speedup vs baseline: 2.5668x; 1.0408x over previous
import functools

import jax
import jax.numpy as jnp
import numpy as np
from jax import lax
from jax.experimental import pallas as pl
from jax.experimental.pallas import tpu as pltpu
from jax.experimental.pallas import tpu_sc as plsc

B = 16
N = 2048
S = 512
K = 32
CIN = 128
DTAB = 256
R2 = np.float32(0.2 ** 2)
F32 = jnp.float32
HI = jax.lax.Precision.HIGHEST


def _fps_kernel(xyz_ref, out_ref):
    xq = xyz_ref[0]
    yq = xyz_ref[1]
    zq = xyz_ref[2]
    iota_n = lax.broadcasted_iota(jnp.int32, (B, N), 1).astype(F32)
    iota_s = lax.broadcasted_iota(jnp.int32, (B, S), 1).astype(F32)

    def body(i, carry):
        dist, far, ax, ay, az = carry
        sel = iota_n == far
        cx = jnp.sum(jnp.where(sel, xq, 0.0), axis=1, keepdims=True)
        cy = jnp.sum(jnp.where(sel, yq, 0.0), axis=1, keepdims=True)
        cz = jnp.sum(jnp.where(sel, zq, 0.0), axis=1, keepdims=True)
        i_f = i.astype(F32)
        ax = jnp.where(iota_s == i_f, cx, ax)
        ay = jnp.where(iota_s == i_f, cy, ay)
        az = jnp.where(iota_s == i_f, cz, az)
        dx = xq - cx
        dy = yq - cy
        dz = zq - cz
        d = dx * dx + dy * dy + dz * dz
        dist = jnp.minimum(dist, d)
        m = jnp.max(dist, axis=1, keepdims=True)
        far = jnp.min(jnp.where(dist == m, iota_n, float(N)), axis=1,
                      keepdims=True)
        return dist, far, ax, ay, az

    dist0 = jnp.full((B, N), 1e10, dtype=F32)
    far0 = jnp.zeros((B, 1), dtype=F32)
    acc0 = jnp.zeros((B, S), dtype=F32)
    _, _, ax, ay, az = lax.fori_loop(0, S, body,
                                     (dist0, far0, acc0, acc0, acc0))
    out_ref[0] = ax
    out_ref[1] = ay
    out_ref[2] = az


def _fps_call(xyz_t):
    return pl.pallas_call(
        _fps_kernel,
        out_shape=jax.ShapeDtypeStruct((3, B, S), F32),
    )(xyz_t)


S_BLK = 64
_CHUNK = 256


def _bq_kernel(xyz_ref, nxyz_ref, out_ref):
    b = pl.program_id(0)
    xq = xyz_ref[0, 0:1, :]
    yq = xyz_ref[0, 1:2, :]
    zq = xyz_ref[0, 2:3, :]
    nb = nxyz_ref[0]
    ax = nb[:, 0:1]
    ay = nb[:, 1:2]
    az = nb[:, 2:3]
    dot = lax.dot_general(nb.astype(jnp.bfloat16),
                          xyz_ref[0].astype(jnp.bfloat16),
                          (((1,), (0,)), ((), ())),
                          preferred_element_type=F32)
    ns = ax * ax + ay * ay + az * az
    nx = xq * xq + yq * yq + zq * zq
    sqr = (ns + nx) - 2.0 * dot
    mask = jnp.where(sqr <= R2, 1.0, 0.0)

    nch = N // _CHUNK
    tri = (lax.broadcasted_iota(jnp.int32, (_CHUNK, _CHUNK), 0)
           <= lax.broadcasted_iota(jnp.int32, (_CHUNK, _CHUNK), 1))
    lcnt = lax.dot_general(
        mask.reshape(S_BLK * nch, _CHUNK).astype(jnp.bfloat16),
        tri.astype(jnp.bfloat16), (((1,), (0,)), ((), ())),
        preferred_element_type=F32).reshape(S_BLK, nch, _CHUNK)
    ctot = lcnt[:, :, _CHUNK - 1:_CHUNK].reshape(S_BLK, nch)
    csum = ctot
    sh = 1
    while sh < nch:
        rolled = jnp.roll(csum, sh, axis=1)
        lanec = lax.broadcasted_iota(jnp.int32, (S_BLK, nch), 1)
        csum = csum + jnp.where(lanec >= sh, rolled, 0.0)
        sh *= 2
    off = (csum - ctot).reshape(S_BLK, nch, 1)
    cnt = (lcnt + off).reshape(S_BLK, N)

    total = cnt[:, N - 1:N]
    capped = jnp.minimum(cnt, 32.0)
    kv = lax.broadcasted_iota(jnp.int32, (1, K, 1), 1).astype(F32)
    acc = jnp.zeros((S_BLK, K), dtype=F32)
    for c in range(N // _CHUNK):
        sub = capped[:, c * _CHUNK:(c + 1) * _CHUNK]
        cmp = jnp.where(sub[:, None, :] <= kv, 1.0, 0.0)
        acc = acc + jnp.sum(cmp, axis=2)
    krow = lax.broadcasted_iota(jnp.int32, (S_BLK, K), 1).astype(F32)
    first = acc[:, 0:1]
    idx = jnp.where(krow < total, acc, first)
    gidx = idx + b.astype(F32) * float(N)
    out_ref[0] = gidx.astype(jnp.int32)


def _bq_call(xyz_bt, new_xyz):
    return pl.pallas_call(
        _bq_kernel,
        grid=(B, S // S_BLK),
        in_specs=[
            pl.BlockSpec((1, 3, N), lambda b, s: (b, 0, 0)),
            pl.BlockSpec((1, S_BLK, 3), lambda b, s: (b, s, 0)),
        ],
        out_specs=pl.BlockSpec((1, S_BLK, K), lambda b, s: (b, s, 0)),
        out_shape=jax.ShapeDtypeStruct((B, S, K), jnp.int32),
        compiler_params=pltpu.CompilerParams(
            dimension_semantics=("arbitrary", "arbitrary")),
    )(xyz_bt, new_xyz)


_GROWS = B * S * K
_GCH = 128


def _gather_sc(table_hbm, gidx_hbm, out_hbm, idx_v, rows0, rows1, sem0, sem1,
               *, rows_per_worker, n_chunks):
    wid = lax.axis_index("s") * 2 + lax.axis_index("c")
    base = wid * rows_per_worker
    pltpu.sync_copy(gidx_hbm.at[pl.ds(base, rows_per_worker)], idx_v)
    bufs = (rows0, rows1)
    sems = (sem0, sem1)
    pltpu.async_copy(table_hbm.at[idx_v.at[pl.ds(0, _GCH)]], rows0, sem0)

    def pair(p, _):
        for ph in range(2):
            j = p * 2 + ph
            buf = bufs[ph]
            pltpu.make_async_copy(table_hbm.at[idx_v.at[pl.ds(0, _GCH)]],
                                  buf, sems[ph]).wait()

            @pl.when(j + 1 < n_chunks)
            def _():
                nxt = (j + 1) * _GCH
                pltpu.async_copy(
                    table_hbm.at[idx_v.at[pl.ds(nxt, _GCH)]],
                    bufs[1 - ph], sems[1 - ph])

            pltpu.sync_copy(buf, out_hbm.at[pl.ds(base + j * _GCH, _GCH)])
        return 0

    lax.fori_loop(0, n_chunks // 2, pair, 0)


def _gather_call(table, gidx):
    info = plsc.get_sparse_core_info()
    nw = info.num_cores * info.num_subcores
    rows_per_worker = _GROWS // nw
    n_chunks = rows_per_worker // _GCH
    mesh = plsc.VectorSubcoreMesh(core_axis_name="c", subcore_axis_name="s")
    f = functools.partial(_gather_sc, rows_per_worker=rows_per_worker,
                          n_chunks=n_chunks)
    return pl.kernel(
        f,
        mesh=mesh,
        out_type=jax.ShapeDtypeStruct((_GROWS, DTAB), F32),
        scratch_types=[
            pltpu.VMEM((rows_per_worker,), jnp.int32),
            pltpu.VMEM((_GCH, DTAB), F32),
            pltpu.VMEM((_GCH, DTAB), F32),
            pltpu.SemaphoreType.DMA,
            pltpu.SemaphoreType.DMA,
        ],
    )(table, gidx)


LT = 2048
NTILES = (B * S * K) // LT
TPB = (S * K) // LT
NTOT = float(B * S * K)


_SQRT_HALF = np.float32(1.0 / np.sqrt(2.0))


def _gelu(x):
    return 0.5 * x * (1.0 + lax.erf(x * _SQRT_HALF))


def _tbias(te_ref, tw_ref, tb_ref):
    t_act = _gelu(te_ref[0])
    return lax.dot_general(t_act, tw_ref[...],
                           (((1,), (1,)), ((), ()))) + tb_ref[...]


def _bn_apply(x, st_ref, g_ref, b_ref):
    mu = st_ref[0:1, :] * (1.0 / NTOT)
    ex2 = st_ref[1:2, :] * (1.0 / NTOT)
    var = ex2 - mu * mu
    denom = jnp.sqrt(var + 1e-5)
    return ((x - mu) / denom) * g_ref[...] + b_ref[...]


def _acc_stats(st_ref, y):
    @pl.when((pl.program_id(0) == 0) & (pl.program_id(1) == 0))
    def _():
        st_ref[...] = jnp.zeros_like(st_ref)

    st_ref[0:1, :] += jnp.sum(y, axis=0, keepdims=True)
    st_ref[1:2, :] += jnp.sum(y * y, axis=0, keepdims=True)


def _conv1_kernel(g_ref, nx_ref, te_ref, tw_ref, tb_ref, w_ref,
                  y_ref, st_ref):
    feat = g_ref[...]
    w = w_ref[...]
    y = lax.dot_general(feat, w, (((1,), (1,)), ((), ())))
    wxyz = w[:, 0:3]
    corr = lax.dot_general(nx_ref[0], wxyz, (((1,), (1,)), ((), ())),
                           precision=HI)
    y = (y.reshape(LT // K, K, 128) - corr[:, None, :]).reshape(LT, 128)
    y = y + _tbias(te_ref, tw_ref, tb_ref)
    _acc_stats(st_ref, y)
    y_ref[0] = y


def _convn_kernel(x_ref, stp_ref, g_ref, b_ref, te_ref, tw_ref, tb_ref,
                  w_ref, y_ref, st_ref):
    x = _gelu(_bn_apply(x_ref[0], stp_ref, g_ref, b_ref))
    y = lax.dot_general(x, w_ref[...], (((1,), (1,)), ((), ())))
    y = y + _tbias(te_ref, tw_ref, tb_ref)
    _acc_stats(st_ref, y)
    y_ref[0] = y


def _pool_kernel(x_ref, stp_ref, g_ref, b_ref, out_ref):
    x = _gelu(_bn_apply(x_ref[0], stp_ref, g_ref, b_ref))
    out_ref[0] = jnp.max(x.reshape(LT // K, K, 256), axis=1)


def _row_spec(c):
    return pl.BlockSpec((1, LT, c), lambda b, l: (b, l, 0))


def _full_spec(shape):
    nd = len(shape)
    return pl.BlockSpec(shape, lambda b, l: (0,) * nd)


def _te_spec():
    return pl.BlockSpec((1, 1, 256), lambda b, l: (b, 0, 0))


def _params(outs=1):
    sem = ("arbitrary", "arbitrary")
    return dict(compiler_params=pltpu.CompilerParams(
        dimension_semantics=sem))


def _conv1_call(gathered, new_xyz, te, tw, tb, w):
    return pl.pallas_call(
        _conv1_kernel,
        grid=(B, TPB),
        in_specs=[
            pl.BlockSpec((LT, DTAB), lambda b, l: (b * TPB + l, 0)),
            pl.BlockSpec((1, LT // K, 3), lambda b, l: (b, l, 0)),
            _te_spec(),
            _full_spec((128, 256)),
            _full_spec((1, 128)),
            _full_spec((128, DTAB)),
        ],
        out_specs=[_row_spec(128), _full_spec((2, 128))],
        out_shape=[
            jax.ShapeDtypeStruct((B, S * K, 128), F32),
            jax.ShapeDtypeStruct((2, 128), F32),
        ],
        **_params(),
    )(gathered, new_xyz, te, tw, tb, w)


def _convn_call(kfn, x, stp, g, bb, te, tw, tb, w, oc):
    cin = x.shape[-1]
    return pl.pallas_call(
        kfn,
        grid=(B, TPB),
        in_specs=[
            _row_spec(cin),
            _full_spec((2, cin)),
            _full_spec((1, cin)),
            _full_spec((1, cin)),
            _te_spec(),
            _full_spec((oc, 256)),
            _full_spec((1, oc)),
            _full_spec((oc, cin)),
        ],
        out_specs=[_row_spec(oc), _full_spec((2, oc))],
        out_shape=[
            jax.ShapeDtypeStruct((B, S * K, oc), F32),
            jax.ShapeDtypeStruct((2, oc), F32),
        ],
        **_params(),
    )(x, stp, g, bb, te, tw, tb, w)


def _pool_call(x, stp, g, bb):
    return pl.pallas_call(
        _pool_kernel,
        grid=(B, TPB),
        in_specs=[
            _row_spec(256),
            _full_spec((2, 256)),
            _full_spec((1, 256)),
            _full_spec((1, 256)),
        ],
        out_specs=pl.BlockSpec((1, LT // K, 256), lambda b, l: (b, l, 0)),
        out_shape=jax.ShapeDtypeStruct((B, S, 256), F32),
        **_params(),
    )(x, stp, g, bb)


def kernel(xyz, points, t_embed, conv_w_0, time_w_0, time_b_0, bn_g_0,
           bn_b_0, conv_w_1, time_w_1, time_b_1, bn_g_1, bn_b_1, conv_w_2,
           time_w_2, time_b_2, bn_g_2, bn_b_2):
    xyz_t = jnp.transpose(xyz, (2, 0, 1))
    nx3 = _fps_call(xyz_t)
    new_xyz = jnp.transpose(nx3, (1, 2, 0))

    xyz_bt = jnp.transpose(xyz, (0, 2, 1))
    gidx = _bq_call(xyz_bt, new_xyz)
    gidx_flat = gidx.reshape(_GROWS)

    pts_t = jnp.transpose(points, (0, 2, 1))
    table = jnp.concatenate(
        [xyz, pts_t, jnp.zeros((B, N, DTAB - 3 - CIN), F32)],
        axis=-1).reshape(B * N, DTAB)
    gathered = _gather_call(table, gidx_flat)

    te = t_embed.reshape(B, 1, 256)
    w1 = jnp.concatenate([conv_w_0, jnp.zeros((128, DTAB - 131), F32)], 1)
    y1, st1 = _conv1_call(gathered, new_xyz, te, time_w_0,
                          time_b_0.reshape(1, 128), w1)
    y2, st2 = _convn_call(_convn_kernel, y1, st1, bn_g_0.reshape(1, 128),
                          bn_b_0.reshape(1, 128), te, time_w_1,
                          time_b_1.reshape(1, 128), conv_w_1, 128)
    y3, st3 = _convn_call(_convn_kernel, y2, st2, bn_g_1.reshape(1, 128),
                          bn_b_1.reshape(1, 128), te, time_w_2,
                          time_b_2.reshape(1, 256), conv_w_2, 256)
    pooled = _pool_call(y3, st3, bn_g_2.reshape(1, 256),
                        bn_b_2.reshape(1, 256))
    new_points = jnp.transpose(pooled, (0, 2, 1))
    return new_xyz, new_points


# speedup vs baseline: 2.8035x; 1.0922x over previous
import functools

import jax
import jax.numpy as jnp
import numpy as np
from jax import lax
from jax.experimental import pallas as pl
from jax.experimental.pallas import tpu as pltpu
from jax.experimental.pallas import tpu_sc as plsc

B = 16
N = 2048
S = 512
K = 32
CIN = 128
DTAB = 256
R2 = np.float32(0.2 ** 2)
F32 = jnp.float32
HI = jax.lax.Precision.HIGHEST


def _fps_kernel(xyz_ref, out_ref):
    xq = xyz_ref[0]
    yq = xyz_ref[1]
    zq = xyz_ref[2]
    iota_n = lax.broadcasted_iota(jnp.int32, (B, N), 1).astype(F32)
    iota_s = lax.broadcasted_iota(jnp.int32, (B, S), 1).astype(F32)

    def body(i, carry):
        dist, far, ax, ay, az = carry
        sel = iota_n == far
        cx = jnp.sum(jnp.where(sel, xq, 0.0), axis=1, keepdims=True)
        cy = jnp.sum(jnp.where(sel, yq, 0.0), axis=1, keepdims=True)
        cz = jnp.sum(jnp.where(sel, zq, 0.0), axis=1, keepdims=True)
        i_f = i.astype(F32)
        ax = jnp.where(iota_s == i_f, cx, ax)
        ay = jnp.where(iota_s == i_f, cy, ay)
        az = jnp.where(iota_s == i_f, cz, az)
        dx = xq - cx
        dy = yq - cy
        dz = zq - cz
        d = dx * dx + dy * dy + dz * dz
        dist = jnp.minimum(dist, d)
        m = jnp.max(dist, axis=1, keepdims=True)
        far = jnp.min(jnp.where(dist == m, iota_n, float(N)), axis=1,
                      keepdims=True)
        return dist, far, ax, ay, az

    dist0 = jnp.full((B, N), 1e10, dtype=F32)
    far0 = jnp.zeros((B, 1), dtype=F32)
    acc0 = jnp.zeros((B, S), dtype=F32)
    _, _, ax, ay, az = lax.fori_loop(0, S, body,
                                     (dist0, far0, acc0, acc0, acc0))
    out_ref[0] = ax
    out_ref[1] = ay
    out_ref[2] = az


def _fps_call(xyz_t):
    return pl.pallas_call(
        _fps_kernel,
        out_shape=jax.ShapeDtypeStruct((3, B, S), F32),
    )(xyz_t)


S_BLK = 64
_CHUNK = 256


def _bq_kernel(xyz_ref, nxyz_ref, out_ref):
    b = pl.program_id(0)
    xq = xyz_ref[0, 0:1, :]
    yq = xyz_ref[0, 1:2, :]
    zq = xyz_ref[0, 2:3, :]
    nb = nxyz_ref[0]
    ax = nb[:, 0:1]
    ay = nb[:, 1:2]
    az = nb[:, 2:3]
    dot = lax.dot_general(nb.astype(jnp.bfloat16),
                          xyz_ref[0].astype(jnp.bfloat16),
                          (((1,), (0,)), ((), ())),
                          preferred_element_type=F32)
    ns = ax * ax + ay * ay + az * az
    nx = xq * xq + yq * yq + zq * zq
    sqr = (ns + nx) - 2.0 * dot
    mask = jnp.where(sqr <= R2, 1.0, 0.0)

    nch = N // _CHUNK
    tri = (lax.broadcasted_iota(jnp.int32, (_CHUNK, _CHUNK), 0)
           <= lax.broadcasted_iota(jnp.int32, (_CHUNK, _CHUNK), 1))
    lcnt = lax.dot_general(
        mask.reshape(S_BLK * nch, _CHUNK).astype(jnp.bfloat16),
        tri.astype(jnp.bfloat16), (((1,), (0,)), ((), ())),
        preferred_element_type=F32).reshape(S_BLK, nch, _CHUNK)
    ctot = lcnt[:, :, _CHUNK - 1:_CHUNK].reshape(S_BLK, nch)
    csum = ctot
    sh = 1
    while sh < nch:
        rolled = jnp.roll(csum, sh, axis=1)
        lanec = lax.broadcasted_iota(jnp.int32, (S_BLK, nch), 1)
        csum = csum + jnp.where(lanec >= sh, rolled, 0.0)
        sh *= 2
    off = (csum - ctot).reshape(S_BLK, nch, 1)
    cnt = (lcnt + off).reshape(S_BLK, N)

    total = cnt[:, N - 1:N]
    capped = jnp.minimum(cnt, 32.0)
    kv = lax.broadcasted_iota(jnp.int32, (1, K, 1), 1).astype(F32)
    acc = jnp.zeros((S_BLK, K), dtype=F32)
    for c in range(N // _CHUNK):
        sub = capped[:, c * _CHUNK:(c + 1) * _CHUNK]
        cmp = jnp.where(sub[:, None, :] <= kv, 1.0, 0.0)
        acc = acc + jnp.sum(cmp, axis=2)
    krow = lax.broadcasted_iota(jnp.int32, (S_BLK, K), 1).astype(F32)
    first = acc[:, 0:1]
    idx = jnp.where(krow < total, acc, first)
    gidx = idx + b.astype(F32) * float(N)
    out_ref[0] = gidx.astype(jnp.int32)


def _bq_call(xyz_bt, new_xyz):
    nb = xyz_bt.shape[0]
    return pl.pallas_call(
        _bq_kernel,
        grid=(nb, S // S_BLK),
        in_specs=[
            pl.BlockSpec((1, 3, N), lambda b, s: (b, 0, 0)),
            pl.BlockSpec((1, S_BLK, 3), lambda b, s: (b, s, 0)),
        ],
        out_specs=pl.BlockSpec((1, S_BLK, K), lambda b, s: (b, s, 0)),
        out_shape=jax.ShapeDtypeStruct((nb, S, K), jnp.int32),
        compiler_params=pltpu.CompilerParams(
            dimension_semantics=("arbitrary", "arbitrary")),
    )(xyz_bt, new_xyz)


_GROWS = B * S * K
_GCH = 128


def _gather_sc(table_hbm, gidx_hbm, out_hbm, idx_v, rows0, rows1, sem0, sem1,
               *, rows_per_worker, n_chunks):
    wid = lax.axis_index("s") * 2 + lax.axis_index("c")
    base = wid * rows_per_worker
    pltpu.sync_copy(gidx_hbm.at[pl.ds(base, rows_per_worker)], idx_v)
    bufs = (rows0, rows1)
    sems = (sem0, sem1)
    pltpu.async_copy(table_hbm.at[idx_v.at[pl.ds(0, _GCH)]], rows0, sem0)

    def pair(p, _):
        for ph in range(2):
            j = p * 2 + ph
            buf = bufs[ph]
            pltpu.make_async_copy(table_hbm.at[idx_v.at[pl.ds(0, _GCH)]],
                                  buf, sems[ph]).wait()

            @pl.when(j + 1 < n_chunks)
            def _():
                nxt = (j + 1) * _GCH
                pltpu.async_copy(
                    table_hbm.at[idx_v.at[pl.ds(nxt, _GCH)]],
                    bufs[1 - ph], sems[1 - ph])

            pltpu.sync_copy(buf, out_hbm.at[pl.ds(base + j * _GCH, _GCH)])
        return 0

    lax.fori_loop(0, n_chunks // 2, pair, 0)


def _gather_call(table, gidx):
    info = plsc.get_sparse_core_info()
    nw = info.num_cores * info.num_subcores
    nrows = gidx.shape[0]
    rows_per_worker = nrows // nw
    n_chunks = rows_per_worker // _GCH
    mesh = plsc.VectorSubcoreMesh(core_axis_name="c", subcore_axis_name="s")
    f = functools.partial(_gather_sc, rows_per_worker=rows_per_worker,
                          n_chunks=n_chunks)
    return pl.kernel(
        f,
        mesh=mesh,
        out_type=jax.ShapeDtypeStruct((nrows, DTAB), F32),
        scratch_types=[
            pltpu.VMEM((rows_per_worker,), jnp.int32),
            pltpu.VMEM((_GCH, DTAB), F32),
            pltpu.VMEM((_GCH, DTAB), F32),
            pltpu.SemaphoreType.DMA,
            pltpu.SemaphoreType.DMA,
        ],
    )(table, gidx)


LT = 2048
NTILES = (B * S * K) // LT
TPB = (S * K) // LT
NTOT = float(B * S * K)


_SQRT_HALF = np.float32(1.0 / np.sqrt(2.0))


def _gelu(x):
    return 0.5 * x * (1.0 + lax.erf(x * _SQRT_HALF))


def _tbias(te_ref, tw_ref, tb_ref):
    t_act = _gelu(te_ref[0])
    return lax.dot_general(t_act, tw_ref[...],
                           (((1,), (1,)), ((), ()))) + tb_ref[...]


def _bn_apply(x, st_ref, g_ref, b_ref):
    mu = st_ref[0:1, :] * (1.0 / NTOT)
    ex2 = st_ref[1:2, :] * (1.0 / NTOT)
    var = ex2 - mu * mu
    denom = jnp.sqrt(var + 1e-5)
    return ((x - mu) / denom) * g_ref[...] + b_ref[...]


def _acc_stats(st_ref, y):
    @pl.when((pl.program_id(0) == 0) & (pl.program_id(1) == 0))
    def _():
        st_ref[...] = jnp.zeros_like(st_ref)

    st_ref[0:1, :] += jnp.sum(y, axis=0, keepdims=True)
    st_ref[1:2, :] += jnp.sum(y * y, axis=0, keepdims=True)


def _conv1_kernel(g_ref, nx_ref, te_ref, tw_ref, tb_ref, w_ref,
                  y_ref, st_ref):
    feat = g_ref[...]
    w = w_ref[...]
    y = lax.dot_general(feat, w, (((1,), (1,)), ((), ())))
    wxyz = w[:, 0:3]
    corr = lax.dot_general(nx_ref[0], wxyz, (((1,), (1,)), ((), ())),
                           precision=HI)
    y = (y.reshape(LT // K, K, 128) - corr[:, None, :]).reshape(LT, 128)
    y = y + _tbias(te_ref, tw_ref, tb_ref)
    _acc_stats(st_ref, y)
    y_ref[0] = y


def _convn_kernel(x_ref, stp_ref, g_ref, b_ref, te_ref, tw_ref, tb_ref,
                  w_ref, y_ref, st_ref):
    x = _gelu(_bn_apply(x_ref[0], stp_ref, g_ref, b_ref))
    y = lax.dot_general(x, w_ref[...], (((1,), (1,)), ((), ())))
    y = y + _tbias(te_ref, tw_ref, tb_ref)
    _acc_stats(st_ref, y)
    y_ref[0] = y


def _pool_kernel(x_ref, stp_ref, g_ref, b_ref, out_ref):
    x = _gelu(_bn_apply(x_ref[0], stp_ref, g_ref, b_ref))
    out_ref[0] = jnp.max(x.reshape(LT // K, K, 256), axis=1)


def _row_spec(c):
    return pl.BlockSpec((1, LT, c), lambda b, l: (b, l, 0))


def _full_spec(shape):
    nd = len(shape)
    return pl.BlockSpec(shape, lambda b, l: (0,) * nd)


def _te_spec():
    return pl.BlockSpec((1, 1, 256), lambda b, l: (b, 0, 0))


def _params(outs=1):
    sem = ("arbitrary", "arbitrary")
    return dict(compiler_params=pltpu.CompilerParams(
        dimension_semantics=sem))


def _conv1_call(gathered, new_xyz, te, tw, tb, w):
    nb = new_xyz.shape[0]
    return pl.pallas_call(
        _conv1_kernel,
        grid=(nb, TPB),
        in_specs=[
            pl.BlockSpec((LT, DTAB), lambda b, l: (b * TPB + l, 0)),
            pl.BlockSpec((1, LT // K, 3), lambda b, l: (b, l, 0)),
            _te_spec(),
            _full_spec((128, 256)),
            _full_spec((1, 128)),
            _full_spec((128, DTAB)),
        ],
        out_specs=[_row_spec(128), _full_spec((2, 128))],
        out_shape=[
            jax.ShapeDtypeStruct((nb, S * K, 128), F32),
            jax.ShapeDtypeStruct((2, 128), F32),
        ],
        **_params(),
    )(gathered, new_xyz, te, tw, tb, w)


def _convn_call(kfn, x, stp, g, bb, te, tw, tb, w, oc):
    cin = x.shape[-1]
    nb = x.shape[0]
    return pl.pallas_call(
        kfn,
        grid=(nb, TPB),
        in_specs=[
            _row_spec(cin),
            _full_spec((2, cin)),
            _full_spec((1, cin)),
            _full_spec((1, cin)),
            _te_spec(),
            _full_spec((oc, 256)),
            _full_spec((1, oc)),
            _full_spec((oc, cin)),
        ],
        out_specs=[_row_spec(oc), _full_spec((2, oc))],
        out_shape=[
            jax.ShapeDtypeStruct((nb, S * K, oc), F32),
            jax.ShapeDtypeStruct((2, oc), F32),
        ],
        **_params(),
    )(x, stp, g, bb, te, tw, tb, w)


def _pool_call(x, stp, g, bb):
    nb = x.shape[0]
    return pl.pallas_call(
        _pool_kernel,
        grid=(nb, TPB),
        in_specs=[
            _row_spec(256),
            _full_spec((2, 256)),
            _full_spec((1, 256)),
            _full_spec((1, 256)),
        ],
        out_specs=pl.BlockSpec((1, LT // K, 256), lambda b, l: (b, l, 0)),
        out_shape=jax.ShapeDtypeStruct((nb, S, 256), F32),
        **_params(),
    )(x, stp, g, bb)


def kernel(xyz, points, t_embed, conv_w_0, time_w_0, time_b_0, bn_g_0,
           bn_b_0, conv_w_1, time_w_1, time_b_1, bn_g_1, bn_b_1, conv_w_2,
           time_w_2, time_b_2, bn_g_2, bn_b_2):
    xyz_t = jnp.transpose(xyz, (2, 0, 1))
    nx3 = _fps_call(xyz_t)
    new_xyz = jnp.transpose(nx3, (1, 2, 0))

    xyz_bt = jnp.transpose(xyz, (0, 2, 1))
    pts_t = jnp.transpose(points, (0, 2, 1))
    table = jnp.concatenate(
        [xyz, pts_t, jnp.zeros((B, N, DTAB - 3 - CIN), F32)],
        axis=-1).reshape(B * N, DTAB)

    te = t_embed.reshape(B, 1, 256)
    w1 = jnp.concatenate([conv_w_0, jnp.zeros((128, DTAB - 131), F32)], 1)
    tb1 = time_b_0.reshape(1, 128)

    HB = B // 2
    y1s, st1s = [], []
    gathered_h, nxh = [], []
    for h in (0, 1):
        sl = slice(h * HB, (h + 1) * HB)
        gidx = _bq_call(xyz_bt[sl], new_xyz[sl]) + jnp.int32(h * HB * N)
        gathered_h.append(_gather_call(table, gidx.reshape(HB * S * K)))
        nxh.append(new_xyz[sl])
    for h in (0, 1):
        sl = slice(h * HB, (h + 1) * HB)
        y1, st1 = _conv1_call(gathered_h[h], nxh[h], te[sl], time_w_0, tb1, w1)
        y1s.append(y1)
        st1s.append(st1)
    st1 = st1s[0] + st1s[1]

    def layer(kfn, ys, stp, g, bb, tw, tb, w, oc):
        outs, sts = [], []
        for h in (0, 1):
            sl = slice(h * HB, (h + 1) * HB)
            y, st = _convn_call(kfn, ys[h], stp, g, bb, te[sl], tw, tb, w, oc)
            outs.append(y)
            sts.append(st)
        return outs, sts[0] + sts[1]

    y2s, st2 = layer(_convn_kernel, y1s, st1, bn_g_0.reshape(1, 128),
                     bn_b_0.reshape(1, 128), time_w_1,
                     time_b_1.reshape(1, 128), conv_w_1, 128)
    y3s, st3 = layer(_convn_kernel, y2s, st2, bn_g_1.reshape(1, 128),
                     bn_b_1.reshape(1, 128), time_w_2,
                     time_b_2.reshape(1, 256), conv_w_2, 256)
    pooled = jnp.concatenate(
        [_pool_call(y3s[h], st3, bn_g_2.reshape(1, 256),
                    bn_b_2.reshape(1, 256)) for h in (0, 1)], axis=0)
    new_points = jnp.transpose(pooled, (0, 2, 1))
    return new_xyz, new_points


# speedup vs baseline: 3.2018x; 1.1421x over previous
import functools

import jax
import jax.numpy as jnp
import numpy as np
from jax import lax
from jax.experimental import pallas as pl
from jax.experimental.pallas import tpu as pltpu
from jax.experimental.pallas import tpu_sc as plsc

B = 16
N = 2048
S = 512
K = 32
CIN = 128
DTAB = 256
R2 = np.float32(0.2 ** 2)
F32 = jnp.float32
HI = jax.lax.Precision.HIGHEST


def _fps_kernel(xyz_ref, out_ref):
    xq = xyz_ref[0]
    yq = xyz_ref[1]
    zq = xyz_ref[2]
    iota_n = lax.broadcasted_iota(jnp.int32, (B, N), 1).astype(F32)
    iota_s = lax.broadcasted_iota(jnp.int32, (B, S), 1).astype(F32)

    def body(i, carry):
        dist, far, ax, ay, az = carry
        sel = iota_n == far
        cx = jnp.sum(jnp.where(sel, xq, 0.0), axis=1, keepdims=True)
        cy = jnp.sum(jnp.where(sel, yq, 0.0), axis=1, keepdims=True)
        cz = jnp.sum(jnp.where(sel, zq, 0.0), axis=1, keepdims=True)
        i_f = i.astype(F32)
        ax = jnp.where(iota_s == i_f, cx, ax)
        ay = jnp.where(iota_s == i_f, cy, ay)
        az = jnp.where(iota_s == i_f, cz, az)
        dx = xq - cx
        dy = yq - cy
        dz = zq - cz
        d = dx * dx + dy * dy + dz * dz
        dist = jnp.minimum(dist, d)
        m = jnp.max(dist, axis=1, keepdims=True)
        far = jnp.min(jnp.where(dist == m, iota_n, float(N)), axis=1,
                      keepdims=True)
        return dist, far, ax, ay, az

    dist0 = jnp.full((B, N), 1e10, dtype=F32)
    far0 = jnp.zeros((B, 1), dtype=F32)
    acc0 = jnp.zeros((B, S), dtype=F32)
    _, _, ax, ay, az = lax.fori_loop(0, S, body,
                                     (dist0, far0, acc0, acc0, acc0))
    out_ref[0] = ax
    out_ref[1] = ay
    out_ref[2] = az


def _fps_call(xyz_t):
    return pl.pallas_call(
        _fps_kernel,
        out_shape=jax.ShapeDtypeStruct((3, B, S), F32),
    )(xyz_t)


S_BLK = 128
_CHUNK = 256


def _bq_kernel(xyz_ref, nxyz_ref, out_ref):
    b = pl.program_id(0)
    xq = xyz_ref[0, 0:1, :]
    yq = xyz_ref[0, 1:2, :]
    zq = xyz_ref[0, 2:3, :]
    nb = nxyz_ref[0]
    ax = nb[:, 0:1]
    ay = nb[:, 1:2]
    az = nb[:, 2:3]
    dot = lax.dot_general(nb.astype(jnp.bfloat16),
                          xyz_ref[0].astype(jnp.bfloat16),
                          (((1,), (0,)), ((), ())),
                          preferred_element_type=F32)
    ns = ax * ax + ay * ay + az * az
    nx = xq * xq + yq * yq + zq * zq
    sqr = (ns + nx) - 2.0 * dot
    mask = jnp.where(sqr <= R2, 1.0, 0.0)

    nch = N // _CHUNK
    tri = (lax.broadcasted_iota(jnp.int32, (_CHUNK, _CHUNK), 0)
           <= lax.broadcasted_iota(jnp.int32, (_CHUNK, _CHUNK), 1))
    lcnt = lax.dot_general(
        mask.reshape(S_BLK * nch, _CHUNK).astype(jnp.bfloat16),
        tri.astype(jnp.bfloat16), (((1,), (0,)), ((), ())),
        preferred_element_type=F32).reshape(S_BLK, nch, _CHUNK)
    ctot = lcnt[:, :, _CHUNK - 1:_CHUNK].reshape(S_BLK, nch)
    csum = ctot
    sh = 1
    while sh < nch:
        rolled = jnp.roll(csum, sh, axis=1)
        lanec = lax.broadcasted_iota(jnp.int32, (S_BLK, nch), 1)
        csum = csum + jnp.where(lanec >= sh, rolled, 0.0)
        sh *= 2
    off = (csum - ctot).reshape(S_BLK, nch, 1)
    cnt = (lcnt + off).reshape(S_BLK, N)

    total = cnt[:, N - 1:N]
    capped = jnp.minimum(cnt, 32.0)
    kv = lax.broadcasted_iota(jnp.int32, (1, K, 1), 1).astype(F32)
    acc = jnp.zeros((S_BLK, K), dtype=F32)
    for c in range(N // _CHUNK):
        sub = capped[:, c * _CHUNK:(c + 1) * _CHUNK]
        cmp = jnp.where(sub[:, None, :] <= kv, 1.0, 0.0)
        acc = acc + jnp.sum(cmp, axis=2)
    krow = lax.broadcasted_iota(jnp.int32, (S_BLK, K), 1).astype(F32)
    first = acc[:, 0:1]
    idx = jnp.where(krow < total, acc, first)
    gidx = idx + b.astype(F32) * float(N)
    out_ref[0] = gidx.astype(jnp.int32)


def _bq_call(xyz_bt, new_xyz):
    nb = xyz_bt.shape[0]
    return pl.pallas_call(
        _bq_kernel,
        grid=(nb, S // S_BLK),
        in_specs=[
            pl.BlockSpec((1, 3, N), lambda b, s: (b, 0, 0)),
            pl.BlockSpec((1, S_BLK, 3), lambda b, s: (b, s, 0)),
        ],
        out_specs=pl.BlockSpec((1, S_BLK, K), lambda b, s: (b, s, 0)),
        out_shape=jax.ShapeDtypeStruct((nb, S, K), jnp.int32),
        compiler_params=pltpu.CompilerParams(
            dimension_semantics=("arbitrary", "arbitrary")),
    )(xyz_bt, new_xyz)


_GROWS = B * S * K
_GCH = 128


def _gather_sc(table_hbm, gidx_hbm, out_hbm, idx_v, rows0, rows1, sem0, sem1,
               *, rows_per_worker, n_chunks):
    wid = lax.axis_index("s") * 2 + lax.axis_index("c")
    base = wid * rows_per_worker
    pltpu.sync_copy(gidx_hbm.at[pl.ds(base, rows_per_worker)], idx_v)
    bufs = (rows0, rows1)
    sems = (sem0, sem1)
    pltpu.async_copy(table_hbm.at[idx_v.at[pl.ds(0, _GCH)]], rows0, sem0)

    def pair(p, _):
        for ph in range(2):
            j = p * 2 + ph
            buf = bufs[ph]
            pltpu.make_async_copy(table_hbm.at[idx_v.at[pl.ds(0, _GCH)]],
                                  buf, sems[ph]).wait()

            @pl.when(j + 1 < n_chunks)
            def _():
                nxt = (j + 1) * _GCH
                pltpu.async_copy(
                    table_hbm.at[idx_v.at[pl.ds(nxt, _GCH)]],
                    bufs[1 - ph], sems[1 - ph])

            pltpu.sync_copy(buf, out_hbm.at[pl.ds(base + j * _GCH, _GCH)])
        return 0

    lax.fori_loop(0, n_chunks // 2, pair, 0)


def _gather_call(table, gidx):
    info = plsc.get_sparse_core_info()
    nw = info.num_cores * info.num_subcores
    nrows = gidx.shape[0]
    rows_per_worker = nrows // nw
    n_chunks = rows_per_worker // _GCH
    mesh = plsc.VectorSubcoreMesh(core_axis_name="c", subcore_axis_name="s")
    f = functools.partial(_gather_sc, rows_per_worker=rows_per_worker,
                          n_chunks=n_chunks)
    return pl.kernel(
        f,
        mesh=mesh,
        out_type=jax.ShapeDtypeStruct((nrows, DTAB), F32),
        scratch_types=[
            pltpu.VMEM((rows_per_worker,), jnp.int32),
            pltpu.VMEM((_GCH, DTAB), F32),
            pltpu.VMEM((_GCH, DTAB), F32),
            pltpu.SemaphoreType.DMA,
            pltpu.SemaphoreType.DMA,
        ],
    )(table, gidx)


LT = 4096
NTILES = (B * S * K) // LT
TPB = (S * K) // LT
NTOT = float(B * S * K)


_SQRT_HALF = np.float32(1.0 / np.sqrt(2.0))


def _gelu(x):
    return 0.5 * x * (1.0 + lax.erf(x * _SQRT_HALF))


def _tbias(te_ref, tw_ref, tb_ref):
    t_act = _gelu(te_ref[0])
    return lax.dot_general(t_act, tw_ref[...],
                           (((1,), (1,)), ((), ()))) + tb_ref[...]


def _bn_apply(x, st_ref, g_ref, b_ref):
    mu = st_ref[0:1, :] * (1.0 / NTOT)
    ex2 = st_ref[1:2, :] * (1.0 / NTOT)
    var = ex2 - mu * mu
    denom = jnp.sqrt(var + 1e-5)
    return ((x - mu) / denom) * g_ref[...] + b_ref[...]


def _acc_stats(st_ref, y):
    @pl.when((pl.program_id(0) == 0) & (pl.program_id(1) == 0))
    def _():
        st_ref[...] = jnp.zeros_like(st_ref)

    st_ref[0:1, :] += jnp.sum(y, axis=0, keepdims=True)
    st_ref[1:2, :] += jnp.sum(y * y, axis=0, keepdims=True)


def _conv1_kernel(g_ref, nx_ref, te_ref, tw_ref, tb_ref, w_ref,
                  y_ref, st_ref):
    feat = g_ref[...]
    w = w_ref[...]
    y = lax.dot_general(feat, w, (((1,), (1,)), ((), ())))
    wxyz = w[:, 0:3]
    corr = lax.dot_general(nx_ref[0], wxyz, (((1,), (1,)), ((), ())),
                           precision=HI)
    y = (y.reshape(LT // K, K, 128) - corr[:, None, :]).reshape(LT, 128)
    y = y + _tbias(te_ref, tw_ref, tb_ref)
    _acc_stats(st_ref, y)
    y_ref[0] = y


def _convn_kernel(x_ref, stp_ref, g_ref, b_ref, te_ref, tw_ref, tb_ref,
                  w_ref, y_ref, st_ref):
    x = _gelu(_bn_apply(x_ref[0], stp_ref, g_ref, b_ref))
    y = lax.dot_general(x, w_ref[...], (((1,), (1,)), ((), ())))
    y = y + _tbias(te_ref, tw_ref, tb_ref)
    _acc_stats(st_ref, y)
    y_ref[0] = y


def _pool_kernel(x_ref, stp_ref, g_ref, b_ref, out_ref):
    x = _gelu(_bn_apply(x_ref[0], stp_ref, g_ref, b_ref))
    out_ref[0] = jnp.max(x.reshape(LT // K, K, 256), axis=1)


def _row_spec(c):
    return pl.BlockSpec((1, LT, c), lambda b, l: (b, l, 0))


def _full_spec(shape):
    nd = len(shape)
    return pl.BlockSpec(shape, lambda b, l: (0,) * nd)


def _te_spec():
    return pl.BlockSpec((1, 1, 256), lambda b, l: (b, 0, 0))


def _params(outs=1):
    sem = ("arbitrary", "arbitrary")
    return dict(compiler_params=pltpu.CompilerParams(
        dimension_semantics=sem))


def _conv1_call(gathered, new_xyz, te, tw, tb, w):
    nb = new_xyz.shape[0]
    return pl.pallas_call(
        _conv1_kernel,
        grid=(nb, TPB),
        in_specs=[
            pl.BlockSpec((LT, DTAB), lambda b, l: (b * TPB + l, 0)),
            pl.BlockSpec((1, LT // K, 3), lambda b, l: (b, l, 0)),
            _te_spec(),
            _full_spec((128, 256)),
            _full_spec((1, 128)),
            _full_spec((128, DTAB)),
        ],
        out_specs=[_row_spec(128), _full_spec((2, 128))],
        out_shape=[
            jax.ShapeDtypeStruct((nb, S * K, 128), F32),
            jax.ShapeDtypeStruct((2, 128), F32),
        ],
        **_params(),
    )(gathered, new_xyz, te, tw, tb, w)


def _convn_call(kfn, x, stp, g, bb, te, tw, tb, w, oc):
    cin = x.shape[-1]
    nb = x.shape[0]
    return pl.pallas_call(
        kfn,
        grid=(nb, TPB),
        in_specs=[
            _row_spec(cin),
            _full_spec((2, cin)),
            _full_spec((1, cin)),
            _full_spec((1, cin)),
            _te_spec(),
            _full_spec((oc, 256)),
            _full_spec((1, oc)),
            _full_spec((oc, cin)),
        ],
        out_specs=[_row_spec(oc), _full_spec((2, oc))],
        out_shape=[
            jax.ShapeDtypeStruct((nb, S * K, oc), F32),
            jax.ShapeDtypeStruct((2, oc), F32),
        ],
        **_params(),
    )(x, stp, g, bb, te, tw, tb, w)


def _pool_call(x, stp, g, bb):
    nb = x.shape[0]
    return pl.pallas_call(
        _pool_kernel,
        grid=(nb, TPB),
        in_specs=[
            _row_spec(256),
            _full_spec((2, 256)),
            _full_spec((1, 256)),
            _full_spec((1, 256)),
        ],
        out_specs=pl.BlockSpec((1, LT // K, 256), lambda b, l: (b, l, 0)),
        out_shape=jax.ShapeDtypeStruct((nb, S, 256), F32),
        **_params(),
    )(x, stp, g, bb)


def kernel(xyz, points, t_embed, conv_w_0, time_w_0, time_b_0, bn_g_0,
           bn_b_0, conv_w_1, time_w_1, time_b_1, bn_g_1, bn_b_1, conv_w_2,
           time_w_2, time_b_2, bn_g_2, bn_b_2):
    xyz_t = jnp.transpose(xyz, (2, 0, 1))
    nx3 = _fps_call(xyz_t)
    new_xyz = jnp.transpose(nx3, (1, 2, 0))

    xyz_bt = jnp.transpose(xyz, (0, 2, 1))
    pts_t = jnp.transpose(points, (0, 2, 1))
    table = jnp.concatenate(
        [xyz, pts_t, jnp.zeros((B, N, DTAB - 3 - CIN), F32)],
        axis=-1).reshape(B * N, DTAB)

    te = t_embed.reshape(B, 1, 256)
    w1 = jnp.concatenate([conv_w_0, jnp.zeros((128, DTAB - 131), F32)], 1)
    tb1 = time_b_0.reshape(1, 128)

    HB = B // 2
    y1s, st1s = [], []
    gathered_h, nxh = [], []
    for h in (0, 1):
        sl = slice(h * HB, (h + 1) * HB)
        gidx = _bq_call(xyz_bt[sl], new_xyz[sl]) + jnp.int32(h * HB * N)
        gathered_h.append(_gather_call(table, gidx.reshape(HB * S * K)))
        nxh.append(new_xyz[sl])
    for h in (0, 1):
        sl = slice(h * HB, (h + 1) * HB)
        y1, st1 = _conv1_call(gathered_h[h], nxh[h], te[sl], time_w_0, tb1, w1)
        y1s.append(y1)
        st1s.append(st1)
    st1 = st1s[0] + st1s[1]

    def layer(kfn, ys, stp, g, bb, tw, tb, w, oc):
        outs, sts = [], []
        for h in (0, 1):
            sl = slice(h * HB, (h + 1) * HB)
            y, st = _convn_call(kfn, ys[h], stp, g, bb, te[sl], tw, tb, w, oc)
            outs.append(y)
            sts.append(st)
        return outs, sts[0] + sts[1]

    y2s, st2 = layer(_convn_kernel, y1s, st1, bn_g_0.reshape(1, 128),
                     bn_b_0.reshape(1, 128), time_w_1,
                     time_b_1.reshape(1, 128), conv_w_1, 128)
    y3s, st3 = layer(_convn_kernel, y2s, st2, bn_g_1.reshape(1, 128),
                     bn_b_1.reshape(1, 128), time_w_2,
                     time_b_2.reshape(1, 256), conv_w_2, 256)
    pooled = jnp.concatenate(
        [_pool_call(y3s[h], st3, bn_g_2.reshape(1, 256),
                    bn_b_2.reshape(1, 256)) for h in (0, 1)], axis=0)
    new_points = jnp.transpose(pooled, (0, 2, 1))
    return new_xyz, new_points
